# Initial kernel scaffold; baseline (speedup 1.0000x reference)
import jax
import jax.numpy as jnp
from jax.experimental import pallas as pl


def kernel(hidden_states, router_weight, gate_up_proj, down_proj):
    raise NotImplementedError("write your pallas kernel here")


# speedup vs baseline: 1.1331x; 1.1331x over previous
import functools

import jax
import jax.numpy as jnp
from jax.experimental import pallas as pl
from jax.experimental.pallas import tpu as pltpu

_H = 1024
_I = 2048
_E = 8
_T = 4096
_RB = 512
_TB = 256
_J = 2
_IC = _I // _J


def _router_body(x_ref, rw_ref, comb_ref, xb_ref):
    x = x_ref[...]
    rw = rw_ref[...]
    logits = jax.lax.dot_general(x, rw, (((1,), (1,)), ((), ())),
                                 preferred_element_type=jnp.float32)
    idx = jax.lax.broadcasted_iota(jnp.int32, logits.shape, 1)
    m1 = jnp.max(logits, axis=1, keepdims=True)
    a1 = jnp.min(jnp.where(logits == m1, idx, _E), axis=1, keepdims=True)
    l2 = jnp.where(idx == a1, -jnp.inf, logits)
    m2 = jnp.max(l2, axis=1, keepdims=True)
    a2 = jnp.min(jnp.where(l2 == m2, idx, _E), axis=1, keepdims=True)
    t = jnp.exp(m2 - m1)
    wf = 1.0 / (1.0 + t)
    ws = t / (1.0 + t)
    comb_ref[...] = jnp.where(idx == a1, wf, 0.0) + jnp.where(idx == a2, ws, 0.0)
    xb_ref[...] = x.astype(jnp.bfloat16)


def _router(x, router_weight):
    return pl.pallas_call(
        _router_body,
        grid=(_T // _RB,),
        in_specs=[
            pl.BlockSpec((_RB, _H), lambda i: (i, 0)),
            pl.BlockSpec((_E, _H), lambda i: (0, 0)),
        ],
        out_specs=[
            pl.BlockSpec((_RB, _E), lambda i: (i, 0)),
            pl.BlockSpec((_RB, _H), lambda i: (i, 0)),
        ],
        out_shape=[
            jax.ShapeDtypeStruct((_T, _E), jnp.float32),
            jax.ShapeDtypeStruct((_T, _H), jnp.bfloat16),
        ],
    )(x, router_weight)


def _moe_body(xb_ref, ga_ref, gb_ref, d_ref, comb_ref, out_ref, acc_ref, sem):
    e = pl.program_id(0)
    j = pl.program_id(1)
    tb = pl.program_id(2)
    xb = xb_ref[...]
    ga = ga_ref[0].astype(jnp.bfloat16)
    gb = gb_ref[0].astype(jnp.bfloat16)
    dn = d_ref[0].astype(jnp.bfloat16)
    cdims = (((1,), (1,)), ((), ()))
    gate = jax.lax.dot_general(xb, ga, cdims, preferred_element_type=jnp.float32)
    up = jax.lax.dot_general(xb, gb, cdims, preferred_element_type=jnp.float32)
    h = gate * up / (1.0 + jnp.exp(-gate))
    y = jax.lax.dot_general(h.astype(jnp.bfloat16), dn, cdims,
                            preferred_element_type=jnp.float32)
    idx = jax.lax.broadcasted_iota(jnp.int32, (_TB, _E), 1)
    w = jnp.sum(jnp.where(idx == e, comb_ref[...], 0.0), axis=1, keepdims=True)
    contrib = y * w
    rows = pl.ds(tb * _TB, _TB)
    first = jnp.logical_and(e == 0, j == 0)

    @pl.when(first)
    def _():
        acc_ref[rows, :] = contrib

    @pl.when(jnp.logical_not(first))
    def _():
        acc_ref[rows, :] += contrib

    @pl.when(jnp.logical_and(e == _E - 1, j == _J - 1))
    def _():
        cp = pltpu.make_async_copy(acc_ref.at[rows, :], out_ref.at[rows, :], sem)
        cp.start()
        cp.wait()


def _experts(xb, gate_up_proj, down_proj, comb):
    return pl.pallas_call(
        _moe_body,
        grid=(_E, _J, _T // _TB),
        in_specs=[
            pl.BlockSpec((_TB, _H), lambda e, j, t: (t, 0)),
            pl.BlockSpec((1, _IC, _H), lambda e, j, t: (e, j, 0)),
            pl.BlockSpec((1, _IC, _H), lambda e, j, t: (e, j + _J, 0)),
            pl.BlockSpec((1, _H, _IC), lambda e, j, t: (e, 0, j)),
            pl.BlockSpec((_TB, _E), lambda e, j, t: (t, 0)),
        ],
        out_specs=pl.BlockSpec(memory_space=pltpu.MemorySpace.HBM),
        out_shape=jax.ShapeDtypeStruct((_T, _H), jnp.float32),
        scratch_shapes=[pltpu.VMEM((_T, _H), jnp.float32),
                        pltpu.SemaphoreType.DMA],
        compiler_params=pltpu.CompilerParams(
            dimension_semantics=("arbitrary", "arbitrary", "arbitrary"),
        ),
    )(xb, gate_up_proj, gate_up_proj, down_proj, comb)


@jax.jit
def kernel(hidden_states, router_weight, gate_up_proj, down_proj):
    shape = hidden_states.shape
    x = hidden_states.reshape(-1, _H)
    comb, xb = _router(x, router_weight)
    out = _experts(xb, gate_up_proj, down_proj, comb)
    return out.reshape(shape)


# speedup vs baseline: 2.0113x; 1.7751x over previous
import functools

import jax
import jax.numpy as jnp
from jax import lax
from jax.experimental import pallas as pl
from jax.experimental.pallas import tpu as pltpu
from jax.experimental.pallas import tpu_sc as plsc

_H = 1024
_I = 2048
_E = 8
_T = 4096
_BT = 256
_S = _T * 2 + _E * _BT
_NB = _S // _BT
_NC = 2
_NS = 16
_NW = _NC * _NS
_TPW = _T // _NW
_CH = 32
_FB = 512


def _router_body(x_ref, rw_ref, pos0_ref, pos1_ref, w1_ref, w2_ref, bexp_ref):
    x = x_ref[...]
    rw = rw_ref[...]
    cdims = (((1,), (1,)), ((), ()))
    lt = lax.dot_general(rw, x, cdims, preferred_element_type=jnp.float32)
    eidx = lax.broadcasted_iota(jnp.int32, (_E, _T), 0)
    m1 = jnp.max(lt, axis=0, keepdims=True)
    a1 = jnp.min(jnp.where(lt == m1, eidx, _E), axis=0, keepdims=True)
    l2 = jnp.where(eidx == a1, -jnp.inf, lt)
    m2 = jnp.max(l2, axis=0, keepdims=True)
    a2 = jnp.min(jnp.where(l2 == m2, eidx, _E), axis=0, keepdims=True)
    tt = jnp.exp(m2 - m1)
    w1_ref[...] = 1.0 / (1.0 + tt)
    w2_ref[...] = tt / (1.0 + tt)
    oh = jnp.logical_or(eidx == a1, eidx == a2).astype(jnp.float32)
    nch = _T // 128
    c2d = (((1,), (0,)), ((), ()))
    mlow = (lax.broadcasted_iota(jnp.int32, (128, 128), 0)
            < lax.broadcasted_iota(jnp.int32, (128, 128), 1)).astype(jnp.float32)
    parts = []
    for c in range(nch):
        ohc = lax.slice(oh, (0, c * 128), (_E, (c + 1) * 128))
        parts.append(lax.dot_general(ohc, mlow, c2d,
                                     preferred_element_type=jnp.float32))
    localrank = jnp.concatenate(parts, axis=1)
    wsel = (lax.broadcasted_iota(jnp.int32, (_T, nch), 0) // 128
            == lax.broadcasted_iota(jnp.int32, (_T, nch), 1)).astype(jnp.float32)
    cc = lax.dot_general(oh, wsel, c2d,
                         preferred_element_type=jnp.float32)
    wlt = (lax.broadcasted_iota(jnp.int32, (nch, _T), 0)
           < lax.broadcasted_iota(jnp.int32, (nch, _T), 1) // 128
           ).astype(jnp.float32)
    cbex = lax.dot_general(cc, wlt, c2d,
                           preferred_element_type=jnp.float32)
    rank = (localrank + cbex + 0.5).astype(jnp.int32)
    counts = (jnp.sum(cc, axis=1, keepdims=True) + 0.5).astype(jnp.int32)
    used = ((counts + _BT - 1) // _BT).astype(jnp.float32)
    m8 = (lax.broadcasted_iota(jnp.int32, (_E, _E), 1)
          < lax.broadcasted_iota(jnp.int32, (_E, _E), 0)).astype(jnp.float32)
    basef = lax.dot_general(m8, used, c2d,
                            preferred_element_type=jnp.float32)
    basei = (basef + 0.5).astype(jnp.int32) * _BT
    pos_e = basei + rank
    pos0_ref[...] = jnp.sum(jnp.where(eidx == a1, pos_e, 0), axis=0, keepdims=True)
    pos1_ref[...] = jnp.sum(jnp.where(eidx == a2, pos_e, 0), axis=0, keepdims=True)
    bb = basei // _BT
    biota = lax.broadcasted_iota(jnp.int32, (1, 128), 1)
    sidx = lax.broadcasted_iota(jnp.int32, (_E, 1), 0)
    acc = jnp.zeros((1, 128), jnp.int32)
    for e in range(_E):
        be = jnp.sum(jnp.where(sidx == e, bb, 0), axis=0, keepdims=True)
        acc = acc + (biota >= be).astype(jnp.int32)
    bexp_ref[...] = acc - 1


def _router(x, router_weight):
    return pl.pallas_call(
        _router_body,
        out_shape=[
            jax.ShapeDtypeStruct((1, _T), jnp.int32),
            jax.ShapeDtypeStruct((1, _T), jnp.int32),
            jax.ShapeDtypeStruct((1, _T), jnp.float32),
            jax.ShapeDtypeStruct((1, _T), jnp.float32),
            jax.ShapeDtypeStruct((1, 128), jnp.int32),
        ],
    )(x, router_weight)


def _sc_dispatch_body(x_hbm, pos0_hbm, pos1_hbm, xs_hbm, idx_v, rows_v, sem):
    wid = lax.axis_index("s") * _NC + lax.axis_index("c")
    base = wid * _TPW
    for c in range(_TPW // _CH):
        tok0 = base + c * _CH
        pltpu.sync_copy(x_hbm.at[pl.ds(tok0, _CH), :], rows_v)
        pltpu.sync_copy(pos0_hbm.at[pl.ds(tok0, _CH)], idx_v)
        pltpu.async_copy(rows_v, xs_hbm.at[idx_v], sem).wait()
        pltpu.sync_copy(pos1_hbm.at[pl.ds(tok0, _CH)], idx_v)
        pltpu.async_copy(rows_v, xs_hbm.at[idx_v], sem).wait()


def _sc_dispatch(x, pos0, pos1):
    mesh = plsc.VectorSubcoreMesh(core_axis_name="c", subcore_axis_name="s")
    return pl.kernel(
        _sc_dispatch_body,
        out_type=jax.ShapeDtypeStruct((_S, _H), jnp.float32),
        mesh=mesh,
        scratch_types=[
            pltpu.VMEM((_CH,), jnp.int32),
            pltpu.VMEM((_CH, _H), jnp.float32),
            pltpu.SemaphoreType.DMA,
        ],
    )(x, pos0, pos1)


def _gmm_body(be_ref, xs_ref, ga_ref, gu_ref, d_ref, ys_ref):
    del be_ref
    cdims = (((1,), (1,)), ((), ()))
    xb = xs_ref[...].astype(jnp.bfloat16)
    g = lax.dot_general(xb, ga_ref[0], cdims,
                        preferred_element_type=jnp.float32)
    u = lax.dot_general(xb, gu_ref[0], cdims,
                        preferred_element_type=jnp.float32)
    h = g * u / (1.0 + jnp.exp(-g))
    ys_ref[...] = lax.dot_general(h.astype(jnp.bfloat16), d_ref[0], cdims,
                                  preferred_element_type=jnp.float32)


def _gmm(xs, gate_up_proj, down_proj, bexp):
    grid_spec = pltpu.PrefetchScalarGridSpec(
        num_scalar_prefetch=1,
        grid=(_NB,),
        in_specs=[
            pl.BlockSpec((_BT, _H), lambda b, be: (b, 0)),
            pl.BlockSpec((1, _I, _H), lambda b, be: (be[b], 0, 0)),
            pl.BlockSpec((1, _I, _H), lambda b, be: (be[b], 1, 0)),
            pl.BlockSpec((1, _H, _I), lambda b, be: (be[b], 0, 0)),
        ],
        out_specs=pl.BlockSpec((_BT, _H), lambda b, be: (b, 0)),
    )
    return pl.pallas_call(
        _gmm_body,
        grid_spec=grid_spec,
        out_shape=jax.ShapeDtypeStruct((_S, _H), jnp.float32),
        compiler_params=pltpu.CompilerParams(
            dimension_semantics=("arbitrary",),
        ),
    )(bexp, xs, gate_up_proj, gate_up_proj, down_proj)


def _sc_gather_body(ys_hbm, pos0_hbm, pos1_hbm, r0_hbm, r1_hbm,
                    idx_v, rows_v, sem):
    wid = lax.axis_index("s") * _NC + lax.axis_index("c")
    base = wid * _TPW
    for c in range(_TPW // _CH):
        tok0 = base + c * _CH
        pltpu.sync_copy(pos0_hbm.at[pl.ds(tok0, _CH)], idx_v)
        pltpu.async_copy(ys_hbm.at[idx_v], rows_v, sem).wait()
        pltpu.sync_copy(rows_v, r0_hbm.at[pl.ds(tok0, _CH), :])
        pltpu.sync_copy(pos1_hbm.at[pl.ds(tok0, _CH)], idx_v)
        pltpu.async_copy(ys_hbm.at[idx_v], rows_v, sem).wait()
        pltpu.sync_copy(rows_v, r1_hbm.at[pl.ds(tok0, _CH), :])


def _sc_gather(ys, pos0, pos1):
    mesh = plsc.VectorSubcoreMesh(core_axis_name="c", subcore_axis_name="s")
    return pl.kernel(
        _sc_gather_body,
        out_type=[
            jax.ShapeDtypeStruct((_T, _H), jnp.float32),
            jax.ShapeDtypeStruct((_T, _H), jnp.float32),
        ],
        mesh=mesh,
        scratch_types=[
            pltpu.VMEM((_CH,), jnp.int32),
            pltpu.VMEM((_CH, _H), jnp.float32),
            pltpu.SemaphoreType.DMA,
        ],
    )(ys, pos0, pos1)


def _combine_body(r0_ref, r1_ref, w1_ref, w2_ref, out_ref):
    cdims = (((1,), (1,)), ((), ()))
    ri = lax.broadcasted_iota(jnp.int32, (_FB, _FB), 0)
    ci = lax.broadcasted_iota(jnp.int32, (_FB, _FB), 1)
    eye = (ri == ci).astype(jnp.float32)
    w1c = lax.dot_general(eye, w1_ref[...], cdims,
                          preferred_element_type=jnp.float32)
    w2c = lax.dot_general(eye, w2_ref[...], cdims,
                          preferred_element_type=jnp.float32)
    out_ref[...] = r0_ref[...] * w1c + r1_ref[...] * w2c


def _combine(r0, r1, w1, w2):
    return pl.pallas_call(
        _combine_body,
        grid=(_T // _FB,),
        in_specs=[
            pl.BlockSpec((_FB, _H), lambda i: (i, 0)),
            pl.BlockSpec((_FB, _H), lambda i: (i, 0)),
            pl.BlockSpec((1, _FB), lambda i: (0, i)),
            pl.BlockSpec((1, _FB), lambda i: (0, i)),
        ],
        out_specs=pl.BlockSpec((_FB, _H), lambda i: (i, 0)),
        out_shape=jax.ShapeDtypeStruct((_T, _H), jnp.float32),
    )(r0, r1, w1, w2)


@jax.jit
def kernel(hidden_states, router_weight, gate_up_proj, down_proj):
    shape = hidden_states.shape
    x = hidden_states.reshape(-1, _H)
    pos0, pos1, w1, w2, bexp = _router(x, router_weight)
    pos0r = pos0.reshape(_T)
    pos1r = pos1.reshape(_T)
    xs = _sc_dispatch(x, pos0r, pos1r)
    ys = _gmm(xs, gate_up_proj.astype(jnp.bfloat16),
              down_proj.astype(jnp.bfloat16), bexp.reshape(128)[:_NB])
    r0, r1 = _sc_gather(ys, pos0r, pos1r)
    out = _combine(r0, r1, w1, w2)
    return out.reshape(shape)


# speedup vs baseline: 2.4862x; 1.2361x over previous
import functools

import jax
import jax.numpy as jnp
from jax import lax
from jax.experimental import pallas as pl
from jax.experimental.pallas import tpu as pltpu
from jax.experimental.pallas import tpu_sc as plsc

_H = 1024
_I = 2048
_E = 8
_T = 4096
_BT = 256
_S = _T * 2 + _E * _BT
_NB = _S // _BT
_NC = 2
_NS = 16
_NW = _NC * _NS
_TPW = _T // _NW
_CH = 32
_FB = 512


def _router_body(x_ref, rw_ref, pos0_ref, pos1_ref, w1_ref, w2_ref, bexp_ref,
                 bval_ref):
    x = x_ref[...]
    rw = rw_ref[...]
    cdims = (((1,), (1,)), ((), ()))
    lt = lax.dot_general(rw, x, cdims, preferred_element_type=jnp.float32)
    eidx = lax.broadcasted_iota(jnp.int32, (_E, _T), 0)
    m1 = jnp.max(lt, axis=0, keepdims=True)
    a1 = jnp.min(jnp.where(lt == m1, eidx, _E), axis=0, keepdims=True)
    l2 = jnp.where(eidx == a1, -jnp.inf, lt)
    m2 = jnp.max(l2, axis=0, keepdims=True)
    a2 = jnp.min(jnp.where(l2 == m2, eidx, _E), axis=0, keepdims=True)
    tt = jnp.exp(m2 - m1)
    w1_ref[...] = 1.0 / (1.0 + tt)
    w2_ref[...] = tt / (1.0 + tt)
    oh = jnp.logical_or(eidx == a1, eidx == a2).astype(jnp.float32)
    nch = _T // 128
    c2d = (((1,), (0,)), ((), ()))
    mlow = (lax.broadcasted_iota(jnp.int32, (128, 128), 0)
            < lax.broadcasted_iota(jnp.int32, (128, 128), 1)).astype(jnp.float32)
    parts = []
    for c in range(nch):
        ohc = lax.slice(oh, (0, c * 128), (_E, (c + 1) * 128))
        parts.append(lax.dot_general(ohc, mlow, c2d,
                                     preferred_element_type=jnp.float32))
    localrank = jnp.concatenate(parts, axis=1)
    wsel = (lax.broadcasted_iota(jnp.int32, (_T, nch), 0) // 128
            == lax.broadcasted_iota(jnp.int32, (_T, nch), 1)).astype(jnp.float32)
    cc = lax.dot_general(oh, wsel, c2d,
                         preferred_element_type=jnp.float32)
    wlt = (lax.broadcasted_iota(jnp.int32, (nch, _T), 0)
           < lax.broadcasted_iota(jnp.int32, (nch, _T), 1) // 128
           ).astype(jnp.float32)
    cbex = lax.dot_general(cc, wlt, c2d,
                           preferred_element_type=jnp.float32)
    rank = (localrank + cbex + 0.5).astype(jnp.int32)
    counts = (jnp.sum(cc, axis=1, keepdims=True) + 0.5).astype(jnp.int32)
    used = ((counts + _BT - 1) // _BT).astype(jnp.float32)
    m8 = (lax.broadcasted_iota(jnp.int32, (_E, _E), 1)
          < lax.broadcasted_iota(jnp.int32, (_E, _E), 0)).astype(jnp.float32)
    basef = lax.dot_general(m8, used, c2d,
                            preferred_element_type=jnp.float32)
    basei = (basef + 0.5).astype(jnp.int32) * _BT
    pos_e = basei + rank
    pos0_ref[...] = jnp.sum(jnp.where(eidx == a1, pos_e, 0), axis=0, keepdims=True)
    pos1_ref[...] = jnp.sum(jnp.where(eidx == a2, pos_e, 0), axis=0, keepdims=True)
    bb = basei // _BT
    biota = lax.broadcasted_iota(jnp.int32, (1, 128), 1)
    sidx = lax.broadcasted_iota(jnp.int32, (_E, 1), 0)
    acc = jnp.zeros((1, 128), jnp.int32)
    for e in range(_E):
        be = jnp.sum(jnp.where(sidx == e, bb, 0), axis=0, keepdims=True)
        acc = acc + (biota >= be).astype(jnp.int32)
    bexp_ref[...] = acc - 1
    tu = (jnp.sum(used, axis=0, keepdims=True) + 0.5).astype(jnp.int32)
    bval_ref[...] = (biota < tu).astype(jnp.int32)


def _router(x, router_weight):
    return pl.pallas_call(
        _router_body,
        out_shape=[
            jax.ShapeDtypeStruct((1, _T), jnp.int32),
            jax.ShapeDtypeStruct((1, _T), jnp.int32),
            jax.ShapeDtypeStruct((1, _T), jnp.float32),
            jax.ShapeDtypeStruct((1, _T), jnp.float32),
            jax.ShapeDtypeStruct((1, 128), jnp.int32),
            jax.ShapeDtypeStruct((1, 128), jnp.int32),
        ],
    )(x, router_weight)


def _sc_dispatch_body(x_hbm, pos0_hbm, pos1_hbm, xs_hbm, idx_v, rows_v, sem):
    wid = lax.axis_index("s") * _NC + lax.axis_index("c")
    base = wid * _TPW
    for c in range(_TPW // _CH):
        tok0 = base + c * _CH
        pltpu.sync_copy(x_hbm.at[pl.ds(tok0, _CH), :], rows_v)
        pltpu.sync_copy(pos0_hbm.at[pl.ds(tok0, _CH)], idx_v)
        pltpu.async_copy(rows_v, xs_hbm.at[idx_v], sem).wait()
        pltpu.sync_copy(pos1_hbm.at[pl.ds(tok0, _CH)], idx_v)
        pltpu.async_copy(rows_v, xs_hbm.at[idx_v], sem).wait()


def _sc_dispatch(x, pos0, pos1):
    mesh = plsc.VectorSubcoreMesh(core_axis_name="c", subcore_axis_name="s")
    return pl.kernel(
        _sc_dispatch_body,
        out_type=jax.ShapeDtypeStruct((_S, _H), jnp.float32),
        mesh=mesh,
        scratch_types=[
            pltpu.VMEM((_CH,), jnp.int32),
            pltpu.VMEM((_CH, _H), jnp.float32),
            pltpu.SemaphoreType.DMA,
        ],
    )(x, pos0, pos1)


def _gmm_body(be_ref, bv_ref, xs_ref, ga_ref, gu_ref, d_ref, ys_ref):
    del be_ref

    @pl.when(bv_ref[pl.program_id(0)] == 1)
    def _():
        cdims = (((1,), (1,)), ((), ()))
        xb = xs_ref[...].astype(jnp.bfloat16)
        g = lax.dot_general(xb, ga_ref[0].astype(jnp.bfloat16), cdims,
                            preferred_element_type=jnp.float32)
        u = lax.dot_general(xb, gu_ref[0].astype(jnp.bfloat16), cdims,
                            preferred_element_type=jnp.float32)
        h = g * u / (1.0 + jnp.exp(-g))
        ys_ref[...] = lax.dot_general(h.astype(jnp.bfloat16),
                                      d_ref[0].astype(jnp.bfloat16), cdims,
                                      preferred_element_type=jnp.float32)


def _gmm(xs, gate_up_proj, down_proj, bexp, bval):
    grid_spec = pltpu.PrefetchScalarGridSpec(
        num_scalar_prefetch=2,
        grid=(_NB,),
        in_specs=[
            pl.BlockSpec((_BT, _H), lambda b, be, bv: (b, 0)),
            pl.BlockSpec((1, _I, _H), lambda b, be, bv: (be[b], 0, 0)),
            pl.BlockSpec((1, _I, _H), lambda b, be, bv: (be[b], 1, 0)),
            pl.BlockSpec((1, _H, _I), lambda b, be, bv: (be[b], 0, 0)),
        ],
        out_specs=pl.BlockSpec((_BT, _H), lambda b, be, bv: (b, 0)),
    )
    return pl.pallas_call(
        _gmm_body,
        grid_spec=grid_spec,
        out_shape=jax.ShapeDtypeStruct((_S, _H), jnp.float32),
        compiler_params=pltpu.CompilerParams(
            dimension_semantics=("arbitrary",),
        ),
    )(bexp, bval, xs, gate_up_proj, gate_up_proj, down_proj)


def _sc_gather_body(ys_hbm, pos0_hbm, pos1_hbm, r0_hbm, r1_hbm,
                    idx_v, rows_v, sem):
    wid = lax.axis_index("s") * _NC + lax.axis_index("c")
    base = wid * _TPW
    for c in range(_TPW // _CH):
        tok0 = base + c * _CH
        pltpu.sync_copy(pos0_hbm.at[pl.ds(tok0, _CH)], idx_v)
        pltpu.async_copy(ys_hbm.at[idx_v], rows_v, sem).wait()
        pltpu.sync_copy(rows_v, r0_hbm.at[pl.ds(tok0, _CH), :])
        pltpu.sync_copy(pos1_hbm.at[pl.ds(tok0, _CH)], idx_v)
        pltpu.async_copy(ys_hbm.at[idx_v], rows_v, sem).wait()
        pltpu.sync_copy(rows_v, r1_hbm.at[pl.ds(tok0, _CH), :])


def _sc_gather(ys, pos0, pos1):
    mesh = plsc.VectorSubcoreMesh(core_axis_name="c", subcore_axis_name="s")
    return pl.kernel(
        _sc_gather_body,
        out_type=[
            jax.ShapeDtypeStruct((_T, _H), jnp.float32),
            jax.ShapeDtypeStruct((_T, _H), jnp.float32),
        ],
        mesh=mesh,
        scratch_types=[
            pltpu.VMEM((_CH,), jnp.int32),
            pltpu.VMEM((_CH, _H), jnp.float32),
            pltpu.SemaphoreType.DMA,
        ],
    )(ys, pos0, pos1)


def _combine_body(r0_ref, r1_ref, w1_ref, w2_ref, out_ref):
    cdims = (((1,), (1,)), ((), ()))
    ri = lax.broadcasted_iota(jnp.int32, (_FB, _FB), 0)
    ci = lax.broadcasted_iota(jnp.int32, (_FB, _FB), 1)
    eye = (ri == ci).astype(jnp.float32)
    w1c = lax.dot_general(eye, w1_ref[...], cdims,
                          preferred_element_type=jnp.float32)
    w2c = lax.dot_general(eye, w2_ref[...], cdims,
                          preferred_element_type=jnp.float32)
    out_ref[...] = r0_ref[...] * w1c + r1_ref[...] * w2c


def _combine(r0, r1, w1, w2):
    return pl.pallas_call(
        _combine_body,
        grid=(_T // _FB,),
        in_specs=[
            pl.BlockSpec((_FB, _H), lambda i: (i, 0)),
            pl.BlockSpec((_FB, _H), lambda i: (i, 0)),
            pl.BlockSpec((1, _FB), lambda i: (0, i)),
            pl.BlockSpec((1, _FB), lambda i: (0, i)),
        ],
        out_specs=pl.BlockSpec((_FB, _H), lambda i: (i, 0)),
        out_shape=jax.ShapeDtypeStruct((_T, _H), jnp.float32),
    )(r0, r1, w1, w2)


@jax.jit
def kernel(hidden_states, router_weight, gate_up_proj, down_proj):
    shape = hidden_states.shape
    x = hidden_states.reshape(-1, _H)
    pos0, pos1, w1, w2, bexp, bval = _router(x, router_weight)
    pos0r = pos0.reshape(_T)
    pos1r = pos1.reshape(_T)
    xs = _sc_dispatch(x, pos0r, pos1r)
    ys = _gmm(xs, gate_up_proj, down_proj, bexp.reshape(128)[:_NB],
              bval.reshape(128)[:_NB])
    r0, r1 = _sc_gather(ys, pos0r, pos1r)
    out = _combine(r0, r1, w1, w2)
    return out.reshape(shape)


# speedup vs baseline: 2.6793x; 1.0776x over previous
import functools

import jax
import jax.numpy as jnp
from jax import lax
from jax.experimental import pallas as pl
from jax.experimental.pallas import tpu as pltpu
from jax.experimental.pallas import tpu_sc as plsc

_H = 1024
_I = 2048
_E = 8
_T = 4096
_BT = 256
_S = _T * 2 + _E * _BT
_NB = _S // _BT
_NC = 2
_NS = 16
_NW = _NC * _NS
_TPW = _T // _NW
_CH = 32
_FB = 512


def _router_body(x_ref, rw_ref, pos0_ref, pos1_ref, w1_ref, w2_ref, bexp_ref,
                 bval_ref, xb_ref):
    x = x_ref[...]
    rw = rw_ref[...]
    cdims = (((1,), (1,)), ((), ()))
    lt = lax.dot_general(rw, x, cdims, preferred_element_type=jnp.float32)
    eidx = lax.broadcasted_iota(jnp.int32, (_E, _T), 0)
    m1 = jnp.max(lt, axis=0, keepdims=True)
    a1 = jnp.min(jnp.where(lt == m1, eidx, _E), axis=0, keepdims=True)
    l2 = jnp.where(eidx == a1, -jnp.inf, lt)
    m2 = jnp.max(l2, axis=0, keepdims=True)
    a2 = jnp.min(jnp.where(l2 == m2, eidx, _E), axis=0, keepdims=True)
    tt = jnp.exp(m2 - m1)
    w1_ref[...] = 1.0 / (1.0 + tt)
    w2_ref[...] = tt / (1.0 + tt)
    oh = jnp.logical_or(eidx == a1, eidx == a2).astype(jnp.float32)
    nch = _T // 128
    c2d = (((1,), (0,)), ((), ()))
    mlow = (lax.broadcasted_iota(jnp.int32, (128, 128), 0)
            < lax.broadcasted_iota(jnp.int32, (128, 128), 1)).astype(jnp.float32)
    parts = []
    for c in range(nch):
        ohc = lax.slice(oh, (0, c * 128), (_E, (c + 1) * 128))
        parts.append(lax.dot_general(ohc, mlow, c2d,
                                     preferred_element_type=jnp.float32))
    localrank = jnp.concatenate(parts, axis=1)
    wsel = (lax.broadcasted_iota(jnp.int32, (_T, nch), 0) // 128
            == lax.broadcasted_iota(jnp.int32, (_T, nch), 1)).astype(jnp.float32)
    cc = lax.dot_general(oh, wsel, c2d,
                         preferred_element_type=jnp.float32)
    wlt = (lax.broadcasted_iota(jnp.int32, (nch, _T), 0)
           < lax.broadcasted_iota(jnp.int32, (nch, _T), 1) // 128
           ).astype(jnp.float32)
    cbex = lax.dot_general(cc, wlt, c2d,
                           preferred_element_type=jnp.float32)
    rank = (localrank + cbex + 0.5).astype(jnp.int32)
    counts = (jnp.sum(cc, axis=1, keepdims=True) + 0.5).astype(jnp.int32)
    used = ((counts + _BT - 1) // _BT).astype(jnp.float32)
    m8 = (lax.broadcasted_iota(jnp.int32, (_E, _E), 1)
          < lax.broadcasted_iota(jnp.int32, (_E, _E), 0)).astype(jnp.float32)
    basef = lax.dot_general(m8, used, c2d,
                            preferred_element_type=jnp.float32)
    basei = (basef + 0.5).astype(jnp.int32) * _BT
    pos_e = basei + rank
    pos0_ref[...] = jnp.sum(jnp.where(eidx == a1, pos_e, 0), axis=0, keepdims=True)
    pos1_ref[...] = jnp.sum(jnp.where(eidx == a2, pos_e, 0), axis=0, keepdims=True)
    bb = basei // _BT
    biota = lax.broadcasted_iota(jnp.int32, (1, 128), 1)
    sidx = lax.broadcasted_iota(jnp.int32, (_E, 1), 0)
    acc = jnp.zeros((1, 128), jnp.int32)
    for e in range(_E):
        be = jnp.sum(jnp.where(sidx == e, bb, 0), axis=0, keepdims=True)
        acc = acc + (biota >= be).astype(jnp.int32)
    bexp_ref[...] = acc - 1
    tu = (jnp.sum(used, axis=0, keepdims=True) + 0.5).astype(jnp.int32)
    bval_ref[...] = (biota < tu).astype(jnp.int32)
    xb_ref[...] = _pack_bf16(x)


def _pack_bf16(x):
    u = lax.bitcast_convert_type(x, jnp.uint32)
    r = (u + jnp.uint32(0x7FFF) + ((u >> 16) & jnp.uint32(1))) >> 16
    lo = r[:, :_H // 2]
    hi = r[:, _H // 2:]
    return lax.bitcast_convert_type(lo | (hi << 16), jnp.int32)


def _unpack_bf16(xi):
    lo = lax.bitcast_convert_type(xi << 16, jnp.float32)
    hi = lax.bitcast_convert_type(xi & jnp.int32(-65536), jnp.float32)
    return jnp.concatenate([lo, hi], axis=1).astype(jnp.bfloat16)


def _router(x, router_weight):
    return pl.pallas_call(
        _router_body,
        out_shape=[
            jax.ShapeDtypeStruct((1, _T), jnp.int32),
            jax.ShapeDtypeStruct((1, _T), jnp.int32),
            jax.ShapeDtypeStruct((1, _T), jnp.float32),
            jax.ShapeDtypeStruct((1, _T), jnp.float32),
            jax.ShapeDtypeStruct((1, 128), jnp.int32),
            jax.ShapeDtypeStruct((1, 128), jnp.int32),
            jax.ShapeDtypeStruct((_T, _H // 2), jnp.int32),
        ],
    )(x, router_weight)


def _sc_dispatch_body(x_hbm, pos0_hbm, pos1_hbm, xs_hbm, idx_v, rows_v, sem):
    wid = lax.axis_index("s") * _NC + lax.axis_index("c")
    base = wid * _TPW
    for c in range(_TPW // _CH):
        tok0 = base + c * _CH
        pltpu.sync_copy(x_hbm.at[pl.ds(tok0, _CH)], rows_v)
        pltpu.sync_copy(pos0_hbm.at[pl.ds(tok0, _CH)], idx_v)
        pltpu.async_copy(rows_v, xs_hbm.at[idx_v], sem).wait()
        pltpu.sync_copy(pos1_hbm.at[pl.ds(tok0, _CH)], idx_v)
        pltpu.async_copy(rows_v, xs_hbm.at[idx_v], sem).wait()


def _sc_dispatch(xbi, pos0, pos1):
    mesh = plsc.VectorSubcoreMesh(core_axis_name="c", subcore_axis_name="s")
    return pl.kernel(
        _sc_dispatch_body,
        out_type=jax.ShapeDtypeStruct((_S, _H // 2), jnp.int32),
        mesh=mesh,
        scratch_types=[
            pltpu.VMEM((_CH,), jnp.int32),
            pltpu.VMEM((_CH, _H // 2), jnp.int32),
            pltpu.SemaphoreType.DMA,
        ],
    )(xbi, pos0, pos1)


def _gmm_body(be_ref, bv_ref, xs_ref, ga_ref, gu_ref, d_ref, ys_ref):
    del be_ref

    @pl.when(bv_ref[pl.program_id(0)] == 1)
    def _():
        cdims = (((1,), (1,)), ((), ()))
        xb = _unpack_bf16(xs_ref[...])
        g = lax.dot_general(xb, ga_ref[0].astype(jnp.bfloat16), cdims,
                            preferred_element_type=jnp.float32)
        u = lax.dot_general(xb, gu_ref[0].astype(jnp.bfloat16), cdims,
                            preferred_element_type=jnp.float32)
        h = g * u / (1.0 + jnp.exp(-g))
        y = lax.dot_general(h.astype(jnp.bfloat16),
                            d_ref[0].astype(jnp.bfloat16), cdims,
                            preferred_element_type=jnp.float32)
        ys_ref[...] = _pack_bf16(y)


def _gmm(xs, gate_up_proj, down_proj, bexp, bval):
    grid_spec = pltpu.PrefetchScalarGridSpec(
        num_scalar_prefetch=2,
        grid=(_NB,),
        in_specs=[
            pl.BlockSpec((_BT, _H // 2), lambda b, be, bv: (b, 0)),
            pl.BlockSpec((1, _I, _H), lambda b, be, bv: (be[b], 0, 0)),
            pl.BlockSpec((1, _I, _H), lambda b, be, bv: (be[b], 1, 0)),
            pl.BlockSpec((1, _H, _I), lambda b, be, bv: (be[b], 0, 0)),
        ],
        out_specs=pl.BlockSpec((_BT, _H // 2), lambda b, be, bv: (b, 0)),
    )
    return pl.pallas_call(
        _gmm_body,
        grid_spec=grid_spec,
        out_shape=jax.ShapeDtypeStruct((_S, _H // 2), jnp.int32),
        compiler_params=pltpu.CompilerParams(
            dimension_semantics=("arbitrary",),
        ),
    )(bexp, bval, xs, gate_up_proj, gate_up_proj, down_proj)


def _sc_gather_body(ys_hbm, pos0_hbm, pos1_hbm, r0_hbm, r1_hbm,
                    idx_v, rows_v, sem):
    wid = lax.axis_index("s") * _NC + lax.axis_index("c")
    base = wid * _TPW
    for c in range(_TPW // _CH):
        tok0 = base + c * _CH
        pltpu.sync_copy(pos0_hbm.at[pl.ds(tok0, _CH)], idx_v)
        pltpu.async_copy(ys_hbm.at[idx_v], rows_v, sem).wait()
        pltpu.sync_copy(rows_v, r0_hbm.at[pl.ds(tok0, _CH), :])
        pltpu.sync_copy(pos1_hbm.at[pl.ds(tok0, _CH)], idx_v)
        pltpu.async_copy(ys_hbm.at[idx_v], rows_v, sem).wait()
        pltpu.sync_copy(rows_v, r1_hbm.at[pl.ds(tok0, _CH), :])


def _sc_gather(ys, pos0, pos1):
    mesh = plsc.VectorSubcoreMesh(core_axis_name="c", subcore_axis_name="s")
    return pl.kernel(
        _sc_gather_body,
        out_type=[
            jax.ShapeDtypeStruct((_T, _H // 2), jnp.int32),
            jax.ShapeDtypeStruct((_T, _H // 2), jnp.int32),
        ],
        mesh=mesh,
        scratch_types=[
            pltpu.VMEM((_CH,), jnp.int32),
            pltpu.VMEM((_CH, _H // 2), jnp.int32),
            pltpu.SemaphoreType.DMA,
        ],
    )(ys, pos0, pos1)


def _combine_body(r0_ref, r1_ref, w1_ref, w2_ref, out_ref):
    cdims = (((1,), (1,)), ((), ()))
    ri = lax.broadcasted_iota(jnp.int32, (_FB, _FB), 0)
    ci = lax.broadcasted_iota(jnp.int32, (_FB, _FB), 1)
    eye = (ri == ci).astype(jnp.float32)
    w1c = lax.dot_general(eye, w1_ref[...], cdims,
                          preferred_element_type=jnp.float32)
    w2c = lax.dot_general(eye, w2_ref[...], cdims,
                          preferred_element_type=jnp.float32)
    r0 = _unpack_bf16(r0_ref[...]).astype(jnp.float32)
    r1 = _unpack_bf16(r1_ref[...]).astype(jnp.float32)
    out_ref[...] = r0 * w1c + r1 * w2c


def _combine(r0, r1, w1, w2):
    return pl.pallas_call(
        _combine_body,
        grid=(_T // _FB,),
        in_specs=[
            pl.BlockSpec((_FB, _H // 2), lambda i: (i, 0)),
            pl.BlockSpec((_FB, _H // 2), lambda i: (i, 0)),
            pl.BlockSpec((1, _FB), lambda i: (0, i)),
            pl.BlockSpec((1, _FB), lambda i: (0, i)),
        ],
        out_specs=pl.BlockSpec((_FB, _H), lambda i: (i, 0)),
        out_shape=jax.ShapeDtypeStruct((_T, _H), jnp.float32),
    )(r0, r1, w1, w2)


@jax.jit
def kernel(hidden_states, router_weight, gate_up_proj, down_proj):
    shape = hidden_states.shape
    x = hidden_states.reshape(-1, _H)
    pos0, pos1, w1, w2, bexp, bval, xbi = _router(x, router_weight)
    pos0r = pos0.reshape(_T)
    pos1r = pos1.reshape(_T)
    xs = _sc_dispatch(xbi, pos0r, pos1r)
    ys = _gmm(xs, gate_up_proj, down_proj,
              bexp.reshape(128)[:_NB], bval.reshape(128)[:_NB])
    r0, r1 = _sc_gather(ys, pos0r, pos1r)
    out = _combine(r0, r1, w1, w2)
    return out.reshape(shape)


# speedup vs baseline: 2.8532x; 1.0649x over previous
import functools

import jax
import jax.numpy as jnp
from jax import lax
from jax.experimental import pallas as pl
from jax.experimental.pallas import tpu as pltpu
from jax.experimental.pallas import tpu_sc as plsc

_H = 1024
_I = 2048
_E = 8
_T = 4096
_BT = 512
_S = _T * 2 + _E * _BT
_NB = _S // _BT
_NC = 2
_NS = 16
_NW = _NC * _NS
_TPW = _T // _NW
_CH = 32
_FB = 512


def _router_body(x_ref, rw_ref, pos0_ref, pos1_ref, w1_ref, w2_ref, bexp_ref,
                 bval_ref, xb_ref):
    x = x_ref[...]
    rw = rw_ref[...]
    cdims = (((1,), (1,)), ((), ()))
    lt = lax.dot_general(rw, x, cdims, preferred_element_type=jnp.float32)
    eidx = lax.broadcasted_iota(jnp.int32, (_E, _T), 0)
    m1 = jnp.max(lt, axis=0, keepdims=True)
    a1 = jnp.min(jnp.where(lt == m1, eidx, _E), axis=0, keepdims=True)
    l2 = jnp.where(eidx == a1, -jnp.inf, lt)
    m2 = jnp.max(l2, axis=0, keepdims=True)
    a2 = jnp.min(jnp.where(l2 == m2, eidx, _E), axis=0, keepdims=True)
    tt = jnp.exp(m2 - m1)
    w1_ref[...] = 1.0 / (1.0 + tt)
    w2_ref[...] = tt / (1.0 + tt)
    oh = jnp.logical_or(eidx == a1, eidx == a2).astype(jnp.float32)
    nch = _T // 128
    c2d = (((1,), (0,)), ((), ()))
    mlow = (lax.broadcasted_iota(jnp.int32, (128, 128), 0)
            < lax.broadcasted_iota(jnp.int32, (128, 128), 1)).astype(jnp.float32)
    parts = []
    for c in range(nch):
        ohc = lax.slice(oh, (0, c * 128), (_E, (c + 1) * 128))
        parts.append(lax.dot_general(ohc, mlow, c2d,
                                     preferred_element_type=jnp.float32))
    localrank = jnp.concatenate(parts, axis=1)
    wsel = (lax.broadcasted_iota(jnp.int32, (_T, nch), 0) // 128
            == lax.broadcasted_iota(jnp.int32, (_T, nch), 1)).astype(jnp.float32)
    cc = lax.dot_general(oh, wsel, c2d,
                         preferred_element_type=jnp.float32)
    wlt = (lax.broadcasted_iota(jnp.int32, (nch, _T), 0)
           < lax.broadcasted_iota(jnp.int32, (nch, _T), 1) // 128
           ).astype(jnp.float32)
    cbex = lax.dot_general(cc, wlt, c2d,
                           preferred_element_type=jnp.float32)
    rank = (localrank + cbex + 0.5).astype(jnp.int32)
    counts = (jnp.sum(cc, axis=1, keepdims=True) + 0.5).astype(jnp.int32)
    used = ((counts + _BT - 1) // _BT).astype(jnp.float32)
    m8 = (lax.broadcasted_iota(jnp.int32, (_E, _E), 1)
          < lax.broadcasted_iota(jnp.int32, (_E, _E), 0)).astype(jnp.float32)
    basef = lax.dot_general(m8, used, c2d,
                            preferred_element_type=jnp.float32)
    basei = (basef + 0.5).astype(jnp.int32) * _BT
    pos_e = basei + rank
    pos0_ref[...] = jnp.sum(jnp.where(eidx == a1, pos_e, 0), axis=0, keepdims=True)
    pos1_ref[...] = jnp.sum(jnp.where(eidx == a2, pos_e, 0), axis=0, keepdims=True)
    bb = basei // _BT
    biota = lax.broadcasted_iota(jnp.int32, (1, 128), 1)
    sidx = lax.broadcasted_iota(jnp.int32, (_E, 1), 0)
    acc = jnp.zeros((1, 128), jnp.int32)
    for e in range(_E):
        be = jnp.sum(jnp.where(sidx == e, bb, 0), axis=0, keepdims=True)
        acc = acc + (biota >= be).astype(jnp.int32)
    bexp_ref[...] = acc - 1
    tu = (jnp.sum(used, axis=0, keepdims=True) + 0.5).astype(jnp.int32)
    bval_ref[...] = (biota < tu).astype(jnp.int32)
    xb_ref[...] = _pack_bf16(x)


def _pack_bf16(x):
    u = lax.bitcast_convert_type(x, jnp.uint32)
    r = (u + jnp.uint32(0x7FFF) + ((u >> 16) & jnp.uint32(1))) >> 16
    lo = r[:, :_H // 2]
    hi = r[:, _H // 2:]
    return lax.bitcast_convert_type(lo | (hi << 16), jnp.int32)


def _unpack_bf16(xi):
    lo = lax.bitcast_convert_type(xi << 16, jnp.float32)
    hi = lax.bitcast_convert_type(xi & jnp.int32(-65536), jnp.float32)
    return jnp.concatenate([lo, hi], axis=1).astype(jnp.bfloat16)


def _router(x, router_weight):
    return pl.pallas_call(
        _router_body,
        out_shape=[
            jax.ShapeDtypeStruct((1, _T), jnp.int32),
            jax.ShapeDtypeStruct((1, _T), jnp.int32),
            jax.ShapeDtypeStruct((1, _T), jnp.float32),
            jax.ShapeDtypeStruct((1, _T), jnp.float32),
            jax.ShapeDtypeStruct((1, 128), jnp.int32),
            jax.ShapeDtypeStruct((1, 128), jnp.int32),
            jax.ShapeDtypeStruct((_T, _H // 2), jnp.int32),
        ],
    )(x, router_weight)


def _sc_dispatch_body(x_hbm, pos0_hbm, pos1_hbm, xs_hbm, idx_v, rows_v, sem):
    wid = lax.axis_index("s") * _NC + lax.axis_index("c")
    base = wid * _TPW
    for c in range(_TPW // _CH):
        tok0 = base + c * _CH
        pltpu.sync_copy(x_hbm.at[pl.ds(tok0, _CH)], rows_v)
        pltpu.sync_copy(pos0_hbm.at[pl.ds(tok0, _CH)], idx_v)
        pltpu.async_copy(rows_v, xs_hbm.at[idx_v], sem).wait()
        pltpu.sync_copy(pos1_hbm.at[pl.ds(tok0, _CH)], idx_v)
        pltpu.async_copy(rows_v, xs_hbm.at[idx_v], sem).wait()


def _sc_dispatch(xbi, pos0, pos1):
    mesh = plsc.VectorSubcoreMesh(core_axis_name="c", subcore_axis_name="s")
    return pl.kernel(
        _sc_dispatch_body,
        out_type=jax.ShapeDtypeStruct((_S, _H // 2), jnp.int32),
        mesh=mesh,
        scratch_types=[
            pltpu.VMEM((_CH,), jnp.int32),
            pltpu.VMEM((_CH, _H // 2), jnp.int32),
            pltpu.SemaphoreType.DMA,
        ],
    )(xbi, pos0, pos1)


def _gmm_body(be_ref, bv_ref, xs_ref, ga_ref, gu_ref, d_ref, ys_ref):
    del be_ref

    @pl.when(bv_ref[pl.program_id(0)] == 1)
    def _():
        cdims = (((1,), (1,)), ((), ()))
        xb = _unpack_bf16(xs_ref[...])
        g = lax.dot_general(xb, ga_ref[0].astype(jnp.bfloat16), cdims,
                            preferred_element_type=jnp.float32)
        u = lax.dot_general(xb, gu_ref[0].astype(jnp.bfloat16), cdims,
                            preferred_element_type=jnp.float32)
        h = g * u / (1.0 + jnp.exp(-g))
        y = lax.dot_general(h.astype(jnp.bfloat16),
                            d_ref[0].astype(jnp.bfloat16), cdims,
                            preferred_element_type=jnp.float32)
        ys_ref[...] = _pack_bf16(y)


def _gmm(xs, gate_up_proj, down_proj, bexp, bval):
    grid_spec = pltpu.PrefetchScalarGridSpec(
        num_scalar_prefetch=2,
        grid=(_NB,),
        in_specs=[
            pl.BlockSpec((_BT, _H // 2), lambda b, be, bv: (b, 0)),
            pl.BlockSpec((1, _I, _H), lambda b, be, bv: (be[b], 0, 0)),
            pl.BlockSpec((1, _I, _H), lambda b, be, bv: (be[b], 1, 0)),
            pl.BlockSpec((1, _H, _I), lambda b, be, bv: (be[b], 0, 0)),
        ],
        out_specs=pl.BlockSpec((_BT, _H // 2), lambda b, be, bv: (b, 0)),
    )
    return pl.pallas_call(
        _gmm_body,
        grid_spec=grid_spec,
        out_shape=jax.ShapeDtypeStruct((_S, _H // 2), jnp.int32),
        compiler_params=pltpu.CompilerParams(
            dimension_semantics=("arbitrary",),
        ),
    )(bexp, bval, xs, gate_up_proj, gate_up_proj, down_proj)


def _sc_gather_body(ys_hbm, pos0_hbm, pos1_hbm, r0_hbm, r1_hbm,
                    idx_v, rows_v, sem):
    wid = lax.axis_index("s") * _NC + lax.axis_index("c")
    base = wid * _TPW
    for c in range(_TPW // _CH):
        tok0 = base + c * _CH
        pltpu.sync_copy(pos0_hbm.at[pl.ds(tok0, _CH)], idx_v)
        pltpu.async_copy(ys_hbm.at[idx_v], rows_v, sem).wait()
        pltpu.sync_copy(rows_v, r0_hbm.at[pl.ds(tok0, _CH), :])
        pltpu.sync_copy(pos1_hbm.at[pl.ds(tok0, _CH)], idx_v)
        pltpu.async_copy(ys_hbm.at[idx_v], rows_v, sem).wait()
        pltpu.sync_copy(rows_v, r1_hbm.at[pl.ds(tok0, _CH), :])


def _sc_gather(ys, pos0, pos1):
    mesh = plsc.VectorSubcoreMesh(core_axis_name="c", subcore_axis_name="s")
    return pl.kernel(
        _sc_gather_body,
        out_type=[
            jax.ShapeDtypeStruct((_T, _H // 2), jnp.int32),
            jax.ShapeDtypeStruct((_T, _H // 2), jnp.int32),
        ],
        mesh=mesh,
        scratch_types=[
            pltpu.VMEM((_CH,), jnp.int32),
            pltpu.VMEM((_CH, _H // 2), jnp.int32),
            pltpu.SemaphoreType.DMA,
        ],
    )(ys, pos0, pos1)


def _combine_body(r0_ref, r1_ref, w1_ref, w2_ref, out_ref):
    cdims = (((1,), (1,)), ((), ()))
    ri = lax.broadcasted_iota(jnp.int32, (_FB, _FB), 0)
    ci = lax.broadcasted_iota(jnp.int32, (_FB, _FB), 1)
    eye = (ri == ci).astype(jnp.float32)
    w1c = lax.dot_general(eye, w1_ref[...], cdims,
                          preferred_element_type=jnp.float32)
    w2c = lax.dot_general(eye, w2_ref[...], cdims,
                          preferred_element_type=jnp.float32)
    r0 = _unpack_bf16(r0_ref[...]).astype(jnp.float32)
    r1 = _unpack_bf16(r1_ref[...]).astype(jnp.float32)
    out_ref[...] = r0 * w1c + r1 * w2c


def _combine(r0, r1, w1, w2):
    return pl.pallas_call(
        _combine_body,
        grid=(_T // _FB,),
        in_specs=[
            pl.BlockSpec((_FB, _H // 2), lambda i: (i, 0)),
            pl.BlockSpec((_FB, _H // 2), lambda i: (i, 0)),
            pl.BlockSpec((1, _FB), lambda i: (0, i)),
            pl.BlockSpec((1, _FB), lambda i: (0, i)),
        ],
        out_specs=pl.BlockSpec((_FB, _H), lambda i: (i, 0)),
        out_shape=jax.ShapeDtypeStruct((_T, _H), jnp.float32),
    )(r0, r1, w1, w2)


@jax.jit
def kernel(hidden_states, router_weight, gate_up_proj, down_proj):
    shape = hidden_states.shape
    x = hidden_states.reshape(-1, _H)
    pos0, pos1, w1, w2, bexp, bval, xbi = _router(x, router_weight)
    pos0r = pos0.reshape(_T)
    pos1r = pos1.reshape(_T)
    xs = _sc_dispatch(xbi, pos0r, pos1r)
    ys = _gmm(xs, gate_up_proj, down_proj,
              bexp.reshape(128)[:_NB], bval.reshape(128)[:_NB])
    r0, r1 = _sc_gather(ys, pos0r, pos1r)
    out = _combine(r0, r1, w1, w2)
    return out.reshape(shape)


# speedup vs baseline: 2.9487x; 1.0335x over previous
import functools

import jax
import jax.numpy as jnp
from jax import lax
from jax.experimental import pallas as pl
from jax.experimental.pallas import tpu as pltpu
from jax.experimental.pallas import tpu_sc as plsc

_H = 1024
_I = 2048
_E = 8
_T = 4096
_BT = 512
_S = _T * 2 + _E * _BT
_NB = _S // _BT
_NC = 2
_NS = 16
_NW = _NC * _NS
_TPW = _T // _NW
_CH = 64
_FB = 512


def _router_body(x_ref, rw_ref, pos0_ref, pos1_ref, w1_ref, w2_ref, bexp_ref,
                 bval_ref, xb_ref):
    x = x_ref[...]
    rw = rw_ref[...]
    cdims = (((1,), (1,)), ((), ()))
    lt = lax.dot_general(rw, x, cdims, preferred_element_type=jnp.float32)
    eidx = lax.broadcasted_iota(jnp.int32, (_E, _T), 0)
    m1 = jnp.max(lt, axis=0, keepdims=True)
    a1 = jnp.min(jnp.where(lt == m1, eidx, _E), axis=0, keepdims=True)
    l2 = jnp.where(eidx == a1, -jnp.inf, lt)
    m2 = jnp.max(l2, axis=0, keepdims=True)
    a2 = jnp.min(jnp.where(l2 == m2, eidx, _E), axis=0, keepdims=True)
    tt = jnp.exp(m2 - m1)
    w1_ref[...] = 1.0 / (1.0 + tt)
    w2_ref[...] = tt / (1.0 + tt)
    oh = jnp.logical_or(eidx == a1, eidx == a2).astype(jnp.float32)
    nch = _T // 128
    c2d = (((1,), (0,)), ((), ()))
    mlow = (lax.broadcasted_iota(jnp.int32, (128, 128), 0)
            < lax.broadcasted_iota(jnp.int32, (128, 128), 1)).astype(jnp.float32)
    parts = []
    for c in range(nch):
        ohc = lax.slice(oh, (0, c * 128), (_E, (c + 1) * 128))
        parts.append(lax.dot_general(ohc, mlow, c2d,
                                     preferred_element_type=jnp.float32))
    localrank = jnp.concatenate(parts, axis=1)
    wsel = (lax.broadcasted_iota(jnp.int32, (_T, nch), 0) // 128
            == lax.broadcasted_iota(jnp.int32, (_T, nch), 1)).astype(jnp.float32)
    cc = lax.dot_general(oh, wsel, c2d,
                         preferred_element_type=jnp.float32)
    wlt = (lax.broadcasted_iota(jnp.int32, (nch, _T), 0)
           < lax.broadcasted_iota(jnp.int32, (nch, _T), 1) // 128
           ).astype(jnp.float32)
    cbex = lax.dot_general(cc, wlt, c2d,
                           preferred_element_type=jnp.float32)
    rank = (localrank + cbex + 0.5).astype(jnp.int32)
    counts = (jnp.sum(cc, axis=1, keepdims=True) + 0.5).astype(jnp.int32)
    used = ((counts + _BT - 1) // _BT).astype(jnp.float32)
    m8 = (lax.broadcasted_iota(jnp.int32, (_E, _E), 1)
          < lax.broadcasted_iota(jnp.int32, (_E, _E), 0)).astype(jnp.float32)
    basef = lax.dot_general(m8, used, c2d,
                            preferred_element_type=jnp.float32)
    basei = (basef + 0.5).astype(jnp.int32) * _BT
    pos_e = basei + rank
    pos0_ref[...] = jnp.sum(jnp.where(eidx == a1, pos_e, 0), axis=0, keepdims=True)
    pos1_ref[...] = jnp.sum(jnp.where(eidx == a2, pos_e, 0), axis=0, keepdims=True)
    bb = basei // _BT
    biota = lax.broadcasted_iota(jnp.int32, (1, 128), 1)
    sidx = lax.broadcasted_iota(jnp.int32, (_E, 1), 0)
    acc = jnp.zeros((1, 128), jnp.int32)
    for e in range(_E):
        be = jnp.sum(jnp.where(sidx == e, bb, 0), axis=0, keepdims=True)
        acc = acc + (biota >= be).astype(jnp.int32)
    bexp_ref[...] = acc - 1
    tu = (jnp.sum(used, axis=0, keepdims=True) + 0.5).astype(jnp.int32)
    bval_ref[...] = (biota < tu).astype(jnp.int32)
    xb_ref[...] = _pack_bf16(x)


def _pack_bf16(x):
    u = lax.bitcast_convert_type(x, jnp.uint32)
    r = (u + jnp.uint32(0x7FFF) + ((u >> 16) & jnp.uint32(1))) >> 16
    lo = r[:, :_H // 2]
    hi = r[:, _H // 2:]
    return lax.bitcast_convert_type(lo | (hi << 16), jnp.int32)


def _unpack_bf16(xi):
    lo = lax.bitcast_convert_type(xi << 16, jnp.float32)
    hi = lax.bitcast_convert_type(xi & jnp.int32(-65536), jnp.float32)
    return jnp.concatenate([lo, hi], axis=1).astype(jnp.bfloat16)


def _router(x, router_weight):
    return pl.pallas_call(
        _router_body,
        out_shape=[
            jax.ShapeDtypeStruct((1, _T), jnp.int32),
            jax.ShapeDtypeStruct((1, _T), jnp.int32),
            jax.ShapeDtypeStruct((1, _T), jnp.float32),
            jax.ShapeDtypeStruct((1, _T), jnp.float32),
            jax.ShapeDtypeStruct((1, 128), jnp.int32),
            jax.ShapeDtypeStruct((1, 128), jnp.int32),
            jax.ShapeDtypeStruct((_T, _H // 2), jnp.int32),
        ],
    )(x, router_weight)


def _sc_dispatch_body(x_hbm, pos0_hbm, pos1_hbm, xs_hbm, idx_v, rows_v, sem):
    wid = lax.axis_index("s") * _NC + lax.axis_index("c")
    base = wid * _TPW
    for c in range(_TPW // _CH):
        tok0 = base + c * _CH
        pltpu.sync_copy(x_hbm.at[pl.ds(tok0, _CH)], rows_v)
        pltpu.sync_copy(pos0_hbm.at[pl.ds(tok0, _CH)], idx_v)
        pltpu.async_copy(rows_v, xs_hbm.at[idx_v], sem).wait()
        pltpu.sync_copy(pos1_hbm.at[pl.ds(tok0, _CH)], idx_v)
        pltpu.async_copy(rows_v, xs_hbm.at[idx_v], sem).wait()


def _sc_dispatch(xbi, pos0, pos1):
    mesh = plsc.VectorSubcoreMesh(core_axis_name="c", subcore_axis_name="s")
    return pl.kernel(
        _sc_dispatch_body,
        out_type=jax.ShapeDtypeStruct((_S, _H // 2), jnp.int32),
        mesh=mesh,
        scratch_types=[
            pltpu.VMEM((_CH,), jnp.int32),
            pltpu.VMEM((_CH, _H // 2), jnp.int32),
            pltpu.SemaphoreType.DMA,
        ],
    )(xbi, pos0, pos1)


def _gmm_body(be_ref, bv_ref, xs_ref, ga_ref, gu_ref, d_ref, ys_ref):
    del be_ref

    @pl.when(bv_ref[pl.program_id(0)] == 1)
    def _():
        cdims = (((1,), (1,)), ((), ()))
        xb = _unpack_bf16(xs_ref[...])
        g = lax.dot_general(xb, ga_ref[0].astype(jnp.bfloat16), cdims,
                            preferred_element_type=jnp.float32)
        u = lax.dot_general(xb, gu_ref[0].astype(jnp.bfloat16), cdims,
                            preferred_element_type=jnp.float32)
        h = g * u / (1.0 + jnp.exp(-g))
        y = lax.dot_general(h.astype(jnp.bfloat16),
                            d_ref[0].astype(jnp.bfloat16), cdims,
                            preferred_element_type=jnp.float32)
        ys_ref[...] = _pack_bf16(y)


def _gmm(xs, gate_up_proj, down_proj, bexp, bval):
    grid_spec = pltpu.PrefetchScalarGridSpec(
        num_scalar_prefetch=2,
        grid=(_NB,),
        in_specs=[
            pl.BlockSpec((_BT, _H // 2), lambda b, be, bv: (b, 0)),
            pl.BlockSpec((1, _I, _H), lambda b, be, bv: (be[b], 0, 0)),
            pl.BlockSpec((1, _I, _H), lambda b, be, bv: (be[b], 1, 0)),
            pl.BlockSpec((1, _H, _I), lambda b, be, bv: (be[b], 0, 0)),
        ],
        out_specs=pl.BlockSpec((_BT, _H // 2), lambda b, be, bv: (b, 0)),
    )
    return pl.pallas_call(
        _gmm_body,
        grid_spec=grid_spec,
        out_shape=jax.ShapeDtypeStruct((_S, _H // 2), jnp.int32),
        compiler_params=pltpu.CompilerParams(
            dimension_semantics=("arbitrary",),
        ),
    )(bexp, bval, xs, gate_up_proj, gate_up_proj, down_proj)


def _sc_gather_body(ys_hbm, pos0_hbm, pos1_hbm, r0_hbm, r1_hbm,
                    idx_v, rows_v, sem):
    wid = lax.axis_index("s") * _NC + lax.axis_index("c")
    base = wid * _TPW
    for c in range(_TPW // _CH):
        tok0 = base + c * _CH
        pltpu.sync_copy(pos0_hbm.at[pl.ds(tok0, _CH)], idx_v)
        pltpu.async_copy(ys_hbm.at[idx_v], rows_v, sem).wait()
        pltpu.sync_copy(rows_v, r0_hbm.at[pl.ds(tok0, _CH), :])
        pltpu.sync_copy(pos1_hbm.at[pl.ds(tok0, _CH)], idx_v)
        pltpu.async_copy(ys_hbm.at[idx_v], rows_v, sem).wait()
        pltpu.sync_copy(rows_v, r1_hbm.at[pl.ds(tok0, _CH), :])


def _sc_gather(ys, pos0, pos1):
    mesh = plsc.VectorSubcoreMesh(core_axis_name="c", subcore_axis_name="s")
    return pl.kernel(
        _sc_gather_body,
        out_type=[
            jax.ShapeDtypeStruct((_T, _H // 2), jnp.int32),
            jax.ShapeDtypeStruct((_T, _H // 2), jnp.int32),
        ],
        mesh=mesh,
        scratch_types=[
            pltpu.VMEM((_CH,), jnp.int32),
            pltpu.VMEM((_CH, _H // 2), jnp.int32),
            pltpu.SemaphoreType.DMA,
        ],
    )(ys, pos0, pos1)


def _combine_body(r0_ref, r1_ref, w1_ref, w2_ref, out_ref):
    cdims = (((1,), (1,)), ((), ()))
    ri = lax.broadcasted_iota(jnp.int32, (_FB, _FB), 0)
    ci = lax.broadcasted_iota(jnp.int32, (_FB, _FB), 1)
    eye = (ri == ci).astype(jnp.float32)
    w1c = lax.dot_general(eye, w1_ref[...], cdims,
                          preferred_element_type=jnp.float32)
    w2c = lax.dot_general(eye, w2_ref[...], cdims,
                          preferred_element_type=jnp.float32)
    r0 = _unpack_bf16(r0_ref[...]).astype(jnp.float32)
    r1 = _unpack_bf16(r1_ref[...]).astype(jnp.float32)
    out_ref[...] = r0 * w1c + r1 * w2c


def _combine(r0, r1, w1, w2):
    return pl.pallas_call(
        _combine_body,
        grid=(_T // _FB,),
        in_specs=[
            pl.BlockSpec((_FB, _H // 2), lambda i: (i, 0)),
            pl.BlockSpec((_FB, _H // 2), lambda i: (i, 0)),
            pl.BlockSpec((1, _FB), lambda i: (0, i)),
            pl.BlockSpec((1, _FB), lambda i: (0, i)),
        ],
        out_specs=pl.BlockSpec((_FB, _H), lambda i: (i, 0)),
        out_shape=jax.ShapeDtypeStruct((_T, _H), jnp.float32),
    )(r0, r1, w1, w2)


@jax.jit
def kernel(hidden_states, router_weight, gate_up_proj, down_proj):
    shape = hidden_states.shape
    x = hidden_states.reshape(-1, _H)
    pos0, pos1, w1, w2, bexp, bval, xbi = _router(x, router_weight)
    pos0r = pos0.reshape(_T)
    pos1r = pos1.reshape(_T)
    xs = _sc_dispatch(xbi, pos0r, pos1r)
    ys = _gmm(xs, gate_up_proj, down_proj,
              bexp.reshape(128)[:_NB], bval.reshape(128)[:_NB])
    r0, r1 = _sc_gather(ys, pos0r, pos1r)
    out = _combine(r0, r1, w1, w2)
    return out.reshape(shape)


# speedup vs baseline: 2.9975x; 1.0165x over previous
import functools

import jax
import jax.numpy as jnp
from jax import lax
from jax.experimental import pallas as pl
from jax.experimental.pallas import tpu as pltpu
from jax.experimental.pallas import tpu_sc as plsc

_H = 1024
_I = 2048
_E = 8
_T = 4096
_BT = 512
_S = _T * 2 + _E * _BT
_NB = _S // _BT
_NC = 2
_NS = 16
_NW = _NC * _NS
_TPW = _T // _NW
_CH = 128
_FB = 512


def _router_body(x_ref, rw_ref, pos0_ref, pos1_ref, w1_ref, w2_ref, bexp_ref,
                 bval_ref, xb_ref):
    x = x_ref[...]
    rw = rw_ref[...]
    cdims = (((1,), (1,)), ((), ()))
    lt = lax.dot_general(rw, x, cdims, preferred_element_type=jnp.float32)
    eidx = lax.broadcasted_iota(jnp.int32, (_E, _T), 0)
    m1 = jnp.max(lt, axis=0, keepdims=True)
    a1 = jnp.min(jnp.where(lt == m1, eidx, _E), axis=0, keepdims=True)
    l2 = jnp.where(eidx == a1, -jnp.inf, lt)
    m2 = jnp.max(l2, axis=0, keepdims=True)
    a2 = jnp.min(jnp.where(l2 == m2, eidx, _E), axis=0, keepdims=True)
    tt = jnp.exp(m2 - m1)
    w1_ref[...] = 1.0 / (1.0 + tt)
    w2_ref[...] = tt / (1.0 + tt)
    oh = jnp.logical_or(eidx == a1, eidx == a2).astype(jnp.float32)
    nch = _T // 128
    c2d = (((1,), (0,)), ((), ()))
    mlow = (lax.broadcasted_iota(jnp.int32, (128, 128), 0)
            < lax.broadcasted_iota(jnp.int32, (128, 128), 1)).astype(jnp.float32)
    parts = []
    for c in range(nch):
        ohc = lax.slice(oh, (0, c * 128), (_E, (c + 1) * 128))
        parts.append(lax.dot_general(ohc, mlow, c2d,
                                     preferred_element_type=jnp.float32))
    localrank = jnp.concatenate(parts, axis=1)
    wsel = (lax.broadcasted_iota(jnp.int32, (_T, nch), 0) // 128
            == lax.broadcasted_iota(jnp.int32, (_T, nch), 1)).astype(jnp.float32)
    cc = lax.dot_general(oh, wsel, c2d,
                         preferred_element_type=jnp.float32)
    wlt = (lax.broadcasted_iota(jnp.int32, (nch, _T), 0)
           < lax.broadcasted_iota(jnp.int32, (nch, _T), 1) // 128
           ).astype(jnp.float32)
    cbex = lax.dot_general(cc, wlt, c2d,
                           preferred_element_type=jnp.float32)
    rank = (localrank + cbex + 0.5).astype(jnp.int32)
    counts = (jnp.sum(cc, axis=1, keepdims=True) + 0.5).astype(jnp.int32)
    used = ((counts + _BT - 1) // _BT).astype(jnp.float32)
    m8 = (lax.broadcasted_iota(jnp.int32, (_E, _E), 1)
          < lax.broadcasted_iota(jnp.int32, (_E, _E), 0)).astype(jnp.float32)
    basef = lax.dot_general(m8, used, c2d,
                            preferred_element_type=jnp.float32)
    basei = (basef + 0.5).astype(jnp.int32) * _BT
    pos_e = basei + rank
    pos0_ref[...] = jnp.sum(jnp.where(eidx == a1, pos_e, 0), axis=0, keepdims=True)
    pos1_ref[...] = jnp.sum(jnp.where(eidx == a2, pos_e, 0), axis=0, keepdims=True)
    bb = basei // _BT
    biota = lax.broadcasted_iota(jnp.int32, (1, 128), 1)
    sidx = lax.broadcasted_iota(jnp.int32, (_E, 1), 0)
    acc = jnp.zeros((1, 128), jnp.int32)
    for e in range(_E):
        be = jnp.sum(jnp.where(sidx == e, bb, 0), axis=0, keepdims=True)
        acc = acc + (biota >= be).astype(jnp.int32)
    bexp_ref[...] = acc - 1
    tu = (jnp.sum(used, axis=0, keepdims=True) + 0.5).astype(jnp.int32)
    bval_ref[...] = (biota < tu).astype(jnp.int32)
    xb_ref[...] = _pack_bf16(x)


def _pack_bf16(x):
    u = lax.bitcast_convert_type(x, jnp.uint32)
    r = (u + jnp.uint32(0x7FFF) + ((u >> 16) & jnp.uint32(1))) >> 16
    lo = r[:, :_H // 2]
    hi = r[:, _H // 2:]
    return lax.bitcast_convert_type(lo | (hi << 16), jnp.int32)


def _unpack_bf16(xi):
    lo = lax.bitcast_convert_type(xi << 16, jnp.float32)
    hi = lax.bitcast_convert_type(xi & jnp.int32(-65536), jnp.float32)
    return jnp.concatenate([lo, hi], axis=1).astype(jnp.bfloat16)


def _router(x, router_weight):
    return pl.pallas_call(
        _router_body,
        out_shape=[
            jax.ShapeDtypeStruct((1, _T), jnp.int32),
            jax.ShapeDtypeStruct((1, _T), jnp.int32),
            jax.ShapeDtypeStruct((1, _T), jnp.float32),
            jax.ShapeDtypeStruct((1, _T), jnp.float32),
            jax.ShapeDtypeStruct((1, 128), jnp.int32),
            jax.ShapeDtypeStruct((1, 128), jnp.int32),
            jax.ShapeDtypeStruct((_T, _H // 2), jnp.int32),
        ],
    )(x, router_weight)


def _sc_dispatch_body(x_hbm, pos0_hbm, pos1_hbm, xs_hbm, idx_v, rows_v, sem):
    wid = lax.axis_index("s") * _NC + lax.axis_index("c")
    base = wid * _TPW
    for c in range(_TPW // _CH):
        tok0 = base + c * _CH
        pltpu.sync_copy(x_hbm.at[pl.ds(tok0, _CH)], rows_v)
        pltpu.sync_copy(pos0_hbm.at[pl.ds(tok0, _CH)], idx_v)
        pltpu.async_copy(rows_v, xs_hbm.at[idx_v], sem).wait()
        pltpu.sync_copy(pos1_hbm.at[pl.ds(tok0, _CH)], idx_v)
        pltpu.async_copy(rows_v, xs_hbm.at[idx_v], sem).wait()


def _sc_dispatch(xbi, pos0, pos1):
    mesh = plsc.VectorSubcoreMesh(core_axis_name="c", subcore_axis_name="s")
    return pl.kernel(
        _sc_dispatch_body,
        out_type=jax.ShapeDtypeStruct((_S, _H // 2), jnp.int32),
        mesh=mesh,
        scratch_types=[
            pltpu.VMEM((_CH,), jnp.int32),
            pltpu.VMEM((_CH, _H // 2), jnp.int32),
            pltpu.SemaphoreType.DMA,
        ],
    )(xbi, pos0, pos1)


def _gmm_body(be_ref, bv_ref, xs_ref, ga_ref, gu_ref, d_ref, ys_ref):
    del be_ref

    @pl.when(bv_ref[pl.program_id(0)] == 1)
    def _():
        cdims = (((1,), (1,)), ((), ()))
        xb = _unpack_bf16(xs_ref[...])
        g = lax.dot_general(xb, ga_ref[0].astype(jnp.bfloat16), cdims,
                            preferred_element_type=jnp.float32)
        u = lax.dot_general(xb, gu_ref[0].astype(jnp.bfloat16), cdims,
                            preferred_element_type=jnp.float32)
        h = g * u / (1.0 + jnp.exp(-g))
        y = lax.dot_general(h.astype(jnp.bfloat16),
                            d_ref[0].astype(jnp.bfloat16), cdims,
                            preferred_element_type=jnp.float32)
        ys_ref[...] = _pack_bf16(y)


def _gmm(xs, gate_up_proj, down_proj, bexp, bval):
    grid_spec = pltpu.PrefetchScalarGridSpec(
        num_scalar_prefetch=2,
        grid=(_NB,),
        in_specs=[
            pl.BlockSpec((_BT, _H // 2), lambda b, be, bv: (b, 0)),
            pl.BlockSpec((1, _I, _H), lambda b, be, bv: (be[b], 0, 0)),
            pl.BlockSpec((1, _I, _H), lambda b, be, bv: (be[b], 1, 0)),
            pl.BlockSpec((1, _H, _I), lambda b, be, bv: (be[b], 0, 0)),
        ],
        out_specs=pl.BlockSpec((_BT, _H // 2), lambda b, be, bv: (b, 0)),
    )
    return pl.pallas_call(
        _gmm_body,
        grid_spec=grid_spec,
        out_shape=jax.ShapeDtypeStruct((_S, _H // 2), jnp.int32),
        compiler_params=pltpu.CompilerParams(
            dimension_semantics=("arbitrary",),
        ),
    )(bexp, bval, xs, gate_up_proj, gate_up_proj, down_proj)


def _sc_gather_body(ys_hbm, pos0_hbm, pos1_hbm, r0_hbm, r1_hbm,
                    idx_v, rows_v, sem):
    wid = lax.axis_index("s") * _NC + lax.axis_index("c")
    base = wid * _TPW
    for c in range(_TPW // _CH):
        tok0 = base + c * _CH
        pltpu.sync_copy(pos0_hbm.at[pl.ds(tok0, _CH)], idx_v)
        pltpu.async_copy(ys_hbm.at[idx_v], rows_v, sem).wait()
        pltpu.sync_copy(rows_v, r0_hbm.at[pl.ds(tok0, _CH), :])
        pltpu.sync_copy(pos1_hbm.at[pl.ds(tok0, _CH)], idx_v)
        pltpu.async_copy(ys_hbm.at[idx_v], rows_v, sem).wait()
        pltpu.sync_copy(rows_v, r1_hbm.at[pl.ds(tok0, _CH), :])


def _sc_gather(ys, pos0, pos1):
    mesh = plsc.VectorSubcoreMesh(core_axis_name="c", subcore_axis_name="s")
    return pl.kernel(
        _sc_gather_body,
        out_type=[
            jax.ShapeDtypeStruct((_T, _H // 2), jnp.int32),
            jax.ShapeDtypeStruct((_T, _H // 2), jnp.int32),
        ],
        mesh=mesh,
        scratch_types=[
            pltpu.VMEM((_CH,), jnp.int32),
            pltpu.VMEM((_CH, _H // 2), jnp.int32),
            pltpu.SemaphoreType.DMA,
        ],
    )(ys, pos0, pos1)


def _combine_body(r0_ref, r1_ref, w1_ref, w2_ref, out_ref):
    cdims = (((1,), (1,)), ((), ()))
    ri = lax.broadcasted_iota(jnp.int32, (_FB, _FB), 0)
    ci = lax.broadcasted_iota(jnp.int32, (_FB, _FB), 1)
    eye = (ri == ci).astype(jnp.float32)
    w1c = lax.dot_general(eye, w1_ref[...], cdims,
                          preferred_element_type=jnp.float32)
    w2c = lax.dot_general(eye, w2_ref[...], cdims,
                          preferred_element_type=jnp.float32)
    r0 = _unpack_bf16(r0_ref[...]).astype(jnp.float32)
    r1 = _unpack_bf16(r1_ref[...]).astype(jnp.float32)
    out_ref[...] = r0 * w1c + r1 * w2c


def _combine(r0, r1, w1, w2):
    return pl.pallas_call(
        _combine_body,
        grid=(_T // _FB,),
        in_specs=[
            pl.BlockSpec((_FB, _H // 2), lambda i: (i, 0)),
            pl.BlockSpec((_FB, _H // 2), lambda i: (i, 0)),
            pl.BlockSpec((1, _FB), lambda i: (0, i)),
            pl.BlockSpec((1, _FB), lambda i: (0, i)),
        ],
        out_specs=pl.BlockSpec((_FB, _H), lambda i: (i, 0)),
        out_shape=jax.ShapeDtypeStruct((_T, _H), jnp.float32),
    )(r0, r1, w1, w2)


@jax.jit
def kernel(hidden_states, router_weight, gate_up_proj, down_proj):
    shape = hidden_states.shape
    x = hidden_states.reshape(-1, _H)
    pos0, pos1, w1, w2, bexp, bval, xbi = _router(x, router_weight)
    pos0r = pos0.reshape(_T)
    pos1r = pos1.reshape(_T)
    xs = _sc_dispatch(xbi, pos0r, pos1r)
    ys = _gmm(xs, gate_up_proj, down_proj,
              bexp.reshape(128)[:_NB], bval.reshape(128)[:_NB])
    r0, r1 = _sc_gather(ys, pos0r, pos1r)
    out = _combine(r0, r1, w1, w2)
    return out.reshape(shape)


# speedup vs baseline: 2.9979x; 1.0001x over previous
import jax
import jax.numpy as jnp
from jax import lax
from jax.experimental import pallas as pl
from jax.experimental.pallas import tpu as pltpu
from jax.experimental.pallas import tpu_sc as plsc

_H = 1024
_I = 2048
_E = 8
_T = 4096
_BT = 512
_S = _T * 2 + _E * _BT
_NB = _S // _BT
_NC = 2
_NS = 16
_NW = _NC * _NS
_TPW = _T // _NW
_CH = 128
_FB = 512


def _router_body(x_ref, rw_ref, pos0_ref, pos1_ref, w1_ref, w2_ref, bexp_ref,
                 bval_ref, xb_ref):
    x = x_ref[...]
    rw = rw_ref[...]
    cdims = (((1,), (1,)), ((), ()))
    lt = lax.dot_general(rw, x, cdims, preferred_element_type=jnp.float32)
    eidx = lax.broadcasted_iota(jnp.int32, (_E, _T), 0)
    m1 = jnp.max(lt, axis=0, keepdims=True)
    a1 = jnp.min(jnp.where(lt == m1, eidx, _E), axis=0, keepdims=True)
    l2 = jnp.where(eidx == a1, -jnp.inf, lt)
    m2 = jnp.max(l2, axis=0, keepdims=True)
    a2 = jnp.min(jnp.where(l2 == m2, eidx, _E), axis=0, keepdims=True)
    tt = jnp.exp(m2 - m1)
    w1_ref[...] = 1.0 / (1.0 + tt)
    w2_ref[...] = tt / (1.0 + tt)
    oh = jnp.logical_or(eidx == a1, eidx == a2).astype(jnp.float32)
    nch = _T // 128
    c2d = (((1,), (0,)), ((), ()))
    mlow = (lax.broadcasted_iota(jnp.int32, (128, 128), 0)
            < lax.broadcasted_iota(jnp.int32, (128, 128), 1)).astype(jnp.float32)
    parts = []
    for c in range(nch):
        ohc = lax.slice(oh, (0, c * 128), (_E, (c + 1) * 128))
        parts.append(lax.dot_general(ohc, mlow, c2d,
                                     preferred_element_type=jnp.float32))
    localrank = jnp.concatenate(parts, axis=1)
    wsel = (lax.broadcasted_iota(jnp.int32, (_T, nch), 0) // 128
            == lax.broadcasted_iota(jnp.int32, (_T, nch), 1)).astype(jnp.float32)
    cc = lax.dot_general(oh, wsel, c2d,
                         preferred_element_type=jnp.float32)
    wlt = (lax.broadcasted_iota(jnp.int32, (nch, _T), 0)
           < lax.broadcasted_iota(jnp.int32, (nch, _T), 1) // 128
           ).astype(jnp.float32)
    cbex = lax.dot_general(cc, wlt, c2d,
                           preferred_element_type=jnp.float32)
    rank = (localrank + cbex + 0.5).astype(jnp.int32)
    counts = (jnp.sum(cc, axis=1, keepdims=True) + 0.5).astype(jnp.int32)
    used = ((counts + _BT - 1) // _BT).astype(jnp.float32)
    m8 = (lax.broadcasted_iota(jnp.int32, (_E, _E), 1)
          < lax.broadcasted_iota(jnp.int32, (_E, _E), 0)).astype(jnp.float32)
    basef = lax.dot_general(m8, used, c2d,
                            preferred_element_type=jnp.float32)
    basei = (basef + 0.5).astype(jnp.int32) * _BT
    pos_e = basei + rank
    pos0_ref[...] = jnp.sum(jnp.where(eidx == a1, pos_e, 0), axis=0, keepdims=True)
    pos1_ref[...] = jnp.sum(jnp.where(eidx == a2, pos_e, 0), axis=0, keepdims=True)
    bb = basei // _BT
    biota = lax.broadcasted_iota(jnp.int32, (1, 128), 1)
    sidx = lax.broadcasted_iota(jnp.int32, (_E, 1), 0)
    acc = jnp.zeros((1, 128), jnp.int32)
    for e in range(_E):
        be = jnp.sum(jnp.where(sidx == e, bb, 0), axis=0, keepdims=True)
        acc = acc + (biota >= be).astype(jnp.int32)
    bexp_ref[...] = acc - 1
    tu = (jnp.sum(used, axis=0, keepdims=True) + 0.5).astype(jnp.int32)
    bval_ref[...] = (biota < tu).astype(jnp.int32)
    xb_ref[...] = _pack_bf16(x)


def _pack_bf16(x):
    u = lax.bitcast_convert_type(x, jnp.uint32)
    r = (u + jnp.uint32(0x7FFF) + ((u >> 16) & jnp.uint32(1))) >> 16
    lo = r[:, :_H // 2]
    hi = r[:, _H // 2:]
    return lax.bitcast_convert_type(lo | (hi << 16), jnp.int32)


def _unpack_bf16(xi):
    lo = lax.bitcast_convert_type(xi << 16, jnp.float32)
    hi = lax.bitcast_convert_type(xi & jnp.int32(-65536), jnp.float32)
    return jnp.concatenate([lo, hi], axis=1).astype(jnp.bfloat16)


def _router(x, router_weight):
    return pl.pallas_call(
        _router_body,
        out_shape=[
            jax.ShapeDtypeStruct((1, _T), jnp.int32),
            jax.ShapeDtypeStruct((1, _T), jnp.int32),
            jax.ShapeDtypeStruct((1, _T), jnp.float32),
            jax.ShapeDtypeStruct((1, _T), jnp.float32),
            jax.ShapeDtypeStruct((1, 128), jnp.int32),
            jax.ShapeDtypeStruct((1, 128), jnp.int32),
            jax.ShapeDtypeStruct((_T, _H // 2), jnp.int32),
        ],
    )(x, router_weight)


def _sc_dispatch_body(x_hbm, pos0_hbm, pos1_hbm, xs_hbm, idx_v, rows_v, sem):
    wid = lax.axis_index("s") * _NC + lax.axis_index("c")
    base = wid * _TPW
    for c in range(_TPW // _CH):
        tok0 = base + c * _CH
        pltpu.sync_copy(x_hbm.at[pl.ds(tok0, _CH)], rows_v)
        pltpu.sync_copy(pos0_hbm.at[pl.ds(tok0, _CH)], idx_v)
        pltpu.async_copy(rows_v, xs_hbm.at[idx_v], sem).wait()
        pltpu.sync_copy(pos1_hbm.at[pl.ds(tok0, _CH)], idx_v)
        pltpu.async_copy(rows_v, xs_hbm.at[idx_v], sem).wait()


def _sc_dispatch(xbi, pos0, pos1):
    mesh = plsc.VectorSubcoreMesh(core_axis_name="c", subcore_axis_name="s")
    return pl.kernel(
        _sc_dispatch_body,
        out_type=jax.ShapeDtypeStruct((_S, _H // 2), jnp.int32),
        mesh=mesh,
        scratch_types=[
            pltpu.VMEM((_CH,), jnp.int32),
            pltpu.VMEM((_CH, _H // 2), jnp.int32),
            pltpu.SemaphoreType.DMA,
        ],
    )(xbi, pos0, pos1)


def _gmm_body(be_ref, bv_ref, xs_ref, ga_ref, gu_ref, d_ref, ys_ref):
    del be_ref

    @pl.when(bv_ref[pl.program_id(0)] == 1)
    def _():
        cdims = (((1,), (1,)), ((), ()))
        xb = _unpack_bf16(xs_ref[...])
        g = lax.dot_general(xb, ga_ref[0].astype(jnp.bfloat16), cdims,
                            preferred_element_type=jnp.float32)
        u = lax.dot_general(xb, gu_ref[0].astype(jnp.bfloat16), cdims,
                            preferred_element_type=jnp.float32)
        h = g * u / (1.0 + jnp.exp(-g))
        y = lax.dot_general(h.astype(jnp.bfloat16),
                            d_ref[0].astype(jnp.bfloat16), cdims,
                            preferred_element_type=jnp.float32)
        ys_ref[...] = _pack_bf16(y)


def _gmm(xs, gate_up_proj, down_proj, bexp, bval):
    grid_spec = pltpu.PrefetchScalarGridSpec(
        num_scalar_prefetch=2,
        grid=(_NB,),
        in_specs=[
            pl.BlockSpec((_BT, _H // 2), lambda b, be, bv: (b, 0)),
            pl.BlockSpec((1, _I, _H), lambda b, be, bv: (be[b], 0, 0)),
            pl.BlockSpec((1, _I, _H), lambda b, be, bv: (be[b], 1, 0)),
            pl.BlockSpec((1, _H, _I), lambda b, be, bv: (be[b], 0, 0)),
        ],
        out_specs=pl.BlockSpec((_BT, _H // 2), lambda b, be, bv: (b, 0)),
    )
    return pl.pallas_call(
        _gmm_body,
        grid_spec=grid_spec,
        out_shape=jax.ShapeDtypeStruct((_S, _H // 2), jnp.int32),
        compiler_params=pltpu.CompilerParams(
            dimension_semantics=("arbitrary",),
        ),
    )(bexp, bval, xs, gate_up_proj, gate_up_proj, down_proj)


def _sc_gather_body(ys_hbm, pos0_hbm, pos1_hbm, r0_hbm, r1_hbm,
                    idx_v, rows_v, sem):
    wid = lax.axis_index("s") * _NC + lax.axis_index("c")
    base = wid * _TPW
    for c in range(_TPW // _CH):
        tok0 = base + c * _CH
        pltpu.sync_copy(pos0_hbm.at[pl.ds(tok0, _CH)], idx_v)
        pltpu.async_copy(ys_hbm.at[idx_v], rows_v, sem).wait()
        pltpu.sync_copy(rows_v, r0_hbm.at[pl.ds(tok0, _CH), :])
        pltpu.sync_copy(pos1_hbm.at[pl.ds(tok0, _CH)], idx_v)
        pltpu.async_copy(ys_hbm.at[idx_v], rows_v, sem).wait()
        pltpu.sync_copy(rows_v, r1_hbm.at[pl.ds(tok0, _CH), :])


def _sc_gather(ys, pos0, pos1):
    mesh = plsc.VectorSubcoreMesh(core_axis_name="c", subcore_axis_name="s")
    return pl.kernel(
        _sc_gather_body,
        out_type=[
            jax.ShapeDtypeStruct((_T, _H // 2), jnp.int32),
            jax.ShapeDtypeStruct((_T, _H // 2), jnp.int32),
        ],
        mesh=mesh,
        scratch_types=[
            pltpu.VMEM((_CH,), jnp.int32),
            pltpu.VMEM((_CH, _H // 2), jnp.int32),
            pltpu.SemaphoreType.DMA,
        ],
    )(ys, pos0, pos1)


def _combine_body(r0_ref, r1_ref, w1_ref, w2_ref, out_ref):
    cdims = (((1,), (1,)), ((), ()))
    ri = lax.broadcasted_iota(jnp.int32, (_FB, _FB), 0)
    ci = lax.broadcasted_iota(jnp.int32, (_FB, _FB), 1)
    eye = (ri == ci).astype(jnp.float32)
    w1c = lax.dot_general(eye, w1_ref[...], cdims,
                          preferred_element_type=jnp.float32)
    w2c = lax.dot_general(eye, w2_ref[...], cdims,
                          preferred_element_type=jnp.float32)
    r0 = _unpack_bf16(r0_ref[...]).astype(jnp.float32)
    r1 = _unpack_bf16(r1_ref[...]).astype(jnp.float32)
    out_ref[...] = r0 * w1c + r1 * w2c


def _combine(r0, r1, w1, w2):
    return pl.pallas_call(
        _combine_body,
        grid=(_T // _FB,),
        in_specs=[
            pl.BlockSpec((_FB, _H // 2), lambda i: (i, 0)),
            pl.BlockSpec((_FB, _H // 2), lambda i: (i, 0)),
            pl.BlockSpec((1, _FB), lambda i: (0, i)),
            pl.BlockSpec((1, _FB), lambda i: (0, i)),
        ],
        out_specs=pl.BlockSpec((_FB, _H), lambda i: (i, 0)),
        out_shape=jax.ShapeDtypeStruct((_T, _H), jnp.float32),
    )(r0, r1, w1, w2)


@jax.jit
def kernel(hidden_states, router_weight, gate_up_proj, down_proj):
    shape = hidden_states.shape
    x = hidden_states.reshape(-1, _H)
    pos0, pos1, w1, w2, bexp, bval, xbi = _router(x, router_weight)
    pos0r = pos0.reshape(_T)
    pos1r = pos1.reshape(_T)
    xs = _sc_dispatch(xbi, pos0r, pos1r)
    ys = _gmm(xs, gate_up_proj, down_proj,
              bexp.reshape(128)[:_NB], bval.reshape(128)[:_NB])
    r0, r1 = _sc_gather(ys, pos0r, pos1r)
    out = _combine(r0, r1, w1, w2)
    return out.reshape(shape)
